# Initial kernel scaffold; baseline (speedup 1.0000x reference)
import jax
import jax.numpy as jnp
from jax.experimental import pallas as pl


def kernel(x, edge_index, W1, b1, a1, W2, b2, a2):
    raise NotImplementedError("write your pallas kernel here")


# speedup vs baseline: 8.1668x; 8.1668x over previous
import jax
import jax.numpy as jnp
from jax import lax
from jax.experimental import pallas as pl
from jax.experimental.pallas import tpu as pltpu
from jax.experimental.pallas import tpu_sc as plsc

N = 10000
E = 320000
D = 128

NC = 2
NS = 16
NW = NC * NS

CH = 128
CPW = 80
PAD_ROWS = NW * CPW
E_PAD = PAD_ROWS * CH

N_PAD = 10240
RPT = N_PAD // NS

DEGW = 128

_mesh = plsc.VectorSubcoreMesh(
    core_axis_name="c", subcore_axis_name="s", num_cores=NC, num_subcores=NS)


def _deg_body(dstc_hbm, zrows_hbm, ones_hbm, out_hbm, didx, ones_v, dacc):
    c = lax.axis_index("c")
    s = lax.axis_index("s")
    w = s * NC + c
    base = w * CPW

    pltpu.sync_copy(dstc_hbm.at[pl.ds(base, CPW)], didx)
    pltpu.sync_copy(ones_hbm, ones_v)
    pltpu.sync_copy(zrows_hbm, dacc.at[pl.ds(s * RPT, RPT)])
    plsc.subcore_barrier()

    def body(j, _):
        pltpu.sync_copy(ones_v, dacc.at[didx.at[j]], add=True)
        return 0
    lax.fori_loop(0, CPW, body, 0)

    plsc.subcore_barrier()
    pltpu.sync_copy(dacc.at[pl.ds(s * RPT, RPT)],
                    out_hbm.at[c, pl.ds(s * RPT, RPT)])


_deg_kernel = pl.kernel(
    _deg_body,
    out_type=jax.ShapeDtypeStruct((NC, N_PAD, DEGW), jnp.float32),
    mesh=_mesh,
    scratch_types=[
        pltpu.VMEM((CPW, CH), jnp.int32),
        pltpu.VMEM((CH, DEGW), jnp.float32),
        pltpu.VMEM_SHARED((N_PAD, DEGW), jnp.float32),
    ],
)


def _mp_body(srcc_hbm, dstc_hbm, g_hbm, zrows_hbm, out_hbm,
             sidx, didx, rows, acc, sem):
    c = lax.axis_index("c")
    s = lax.axis_index("s")
    w = s * NC + c
    base = w * CPW

    pltpu.sync_copy(srcc_hbm.at[pl.ds(base, CPW)], sidx)
    pltpu.sync_copy(dstc_hbm.at[pl.ds(base, CPW)], didx)
    pltpu.sync_copy(zrows_hbm, acc.at[pl.ds(s * RPT, RPT)])
    plsc.subcore_barrier()

    def body(j, _):
        pltpu.async_copy(g_hbm.at[sidx.at[j]], rows, sem).wait()
        pltpu.sync_copy(rows, acc.at[didx.at[j]], add=True)
        return 0
    lax.fori_loop(0, CPW, body, 0)

    plsc.subcore_barrier()
    pltpu.sync_copy(acc.at[pl.ds(s * RPT, RPT)],
                    out_hbm.at[c, pl.ds(s * RPT, RPT)])


_mp_kernel = pl.kernel(
    _mp_body,
    out_type=jax.ShapeDtypeStruct((NC, N_PAD, D), jnp.float32),
    mesh=_mesh,
    scratch_types=[
        pltpu.VMEM((CPW, CH), jnp.int32),
        pltpu.VMEM((CPW, CH), jnp.int32),
        pltpu.VMEM((CH, D), jnp.float32),
        pltpu.VMEM_SHARED((N_PAD, D), jnp.float32),
        pltpu.SemaphoreType.DMA,
    ],
)


BN = 1000


def _dinv_of(deg0_ref, deg1_ref):
    deg = deg0_ref[:, 0:1] + deg1_ref[:, 0:1] + 1.0
    return lax.rsqrt(jnp.maximum(deg, 1.0))


def _tc1_body(x_ref, deg0_ref, deg1_ref, w1_ref, g1_ref):
    dinv = _dinv_of(deg0_ref, deg1_ref)
    h = jnp.dot(x_ref[...], w1_ref[...], preferred_element_type=jnp.float32)
    g1_ref[...] = h * dinv


def _tc2_body(a0_ref, a1_ref, g1_ref, deg0_ref, deg1_ref,
              b1_ref, al1_ref, w2_ref, g2_ref):
    dinv = _dinv_of(deg0_ref, deg1_ref)
    pre = dinv * (a0_ref[...] + a1_ref[...] + g1_ref[...]) + b1_ref[...]
    z = jnp.where(pre >= 0, pre, al1_ref[...] * pre)
    h2 = jnp.dot(z, w2_ref[...], preferred_element_type=jnp.float32)
    g2_ref[...] = h2 * dinv


def _tc3_body(a0_ref, a1_ref, g2_ref, deg0_ref, deg1_ref,
              b2_ref, al2_ref, out_ref):
    dinv = _dinv_of(deg0_ref, deg1_ref)
    pre = dinv * (a0_ref[...] + a1_ref[...] + g2_ref[...]) + b2_ref[...]
    out_ref[...] = jnp.where(pre >= 0, pre, al2_ref[...] * pre)


def _row_spec(width):
    return pl.BlockSpec((BN, width), lambda i: (i, 0))


def _padrow_spec(width):
    return None


def _full_spec(shape):
    return pl.BlockSpec(shape, lambda i: tuple(0 for _ in shape))


def kernel(x, edge_index, W1, b1, a1, W2, b2, a2):
    src = edge_index[0]
    dst = edge_index[1]
    pad = E_PAD - E
    srcc = jnp.pad(src, (0, pad)).reshape(PAD_ROWS, CH)
    dstc = jnp.pad(dst, (0, pad), constant_values=N).reshape(PAD_ROWS, CH)

    zdeg = jnp.zeros((RPT, DEGW), jnp.float32)
    zacc = jnp.zeros((RPT, D), jnp.float32)

    ones = jnp.ones((CH, DEGW), jnp.float32)
    degp = _deg_kernel(dstc, zdeg, ones)
    deg0, deg1 = degp[0], degp[1]

    b1r = b1.reshape(1, D)
    a1r = a1.reshape(1, D)
    b2r = b2.reshape(1, D)
    a2r = a2.reshape(1, D)

    grid = (N // BN,)
    g1 = pl.pallas_call(
        _tc1_body,
        grid=grid,
        in_specs=[_row_spec(D), _row_spec(DEGW), _row_spec(DEGW),
                  _full_spec((D, D))],
        out_specs=_row_spec(D),
        out_shape=jax.ShapeDtypeStruct((N, D), jnp.float32),
    )(x, deg0, deg1, W1)

    acc1 = _mp_kernel(srcc, dstc, g1, zacc)

    g2 = pl.pallas_call(
        _tc2_body,
        grid=grid,
        in_specs=[_row_spec(D), _row_spec(D), _row_spec(D),
                  _row_spec(DEGW), _row_spec(DEGW),
                  _full_spec((1, D)), _full_spec((1, D)), _full_spec((D, D))],
        out_specs=_row_spec(D),
        out_shape=jax.ShapeDtypeStruct((N, D), jnp.float32),
    )(acc1[0], acc1[1], g1, deg0, deg1, b1r, a1r, W2)

    acc2 = _mp_kernel(srcc, dstc, g2, zacc)

    out = pl.pallas_call(
        _tc3_body,
        grid=grid,
        in_specs=[_row_spec(D), _row_spec(D), _row_spec(D),
                  _row_spec(DEGW), _row_spec(DEGW),
                  _full_spec((1, D)), _full_spec((1, D))],
        out_specs=_row_spec(D),
        out_shape=jax.ShapeDtypeStruct((N, D), jnp.float32),
    )(acc2[0], acc2[1], g2, deg0, deg1, b2r, a2r)

    return out
